# baseline (speedup 1.0000x reference)
import functools
from typing import Optional

import jax
import jax.numpy as jnp
from jax.experimental import pallas as pl
from jax.experimental.pallas import tpu as pltpu

_K = 16
_GROUPS = 8
_EPS = 1e-5
_INF = float("inf")


def _fps_body(pts_ref, idx_ref, pt_ref, *, n, m, rows):
    x = pts_ref[0, 0]
    y = pts_ref[0, 1]
    z = pts_ref[0, 2]
    fi = (jax.lax.broadcasted_iota(jnp.int32, (rows, 128), 0) * 128
          + jax.lax.broadcasted_iota(jnp.int32, (rows, 128), 1))
    px0 = x[0, 0]
    py0 = y[0, 0]
    pz0 = z[0, 0]
    dx = x - px0
    dy = y - py0
    dz = z - pz0
    d0 = (dx * dx + dy * dy) + dz * dz
    idx_ref[0, 0:1, 0:1] = jnp.zeros((1, 1), jnp.int32)
    pt_ref[0, 0:1, 0:1] = jnp.reshape(px0, (1, 1))
    pt_ref[0, 0:1, 1:2] = jnp.reshape(py0, (1, 1))
    pt_ref[0, 0:1, 2:3] = jnp.reshape(pz0, (1, 1))

    def body(i, d):
        rmax = jnp.max(d)
        nxt = jnp.min(jnp.where(d == rmax, fi, n))
        sel = fi == nxt
        px = jnp.sum(jnp.where(sel, x, 0.0))
        py = jnp.sum(jnp.where(sel, y, 0.0))
        pz = jnp.sum(jnp.where(sel, z, 0.0))
        idx_ref[0, pl.ds(i, 1), 0:1] = jnp.reshape(nxt, (1, 1))
        pt_ref[0, pl.ds(i, 1), 0:1] = jnp.reshape(px, (1, 1))
        pt_ref[0, pl.ds(i, 1), 1:2] = jnp.reshape(py, (1, 1))
        pt_ref[0, pl.ds(i, 1), 2:3] = jnp.reshape(pz, (1, 1))
        ddx = x - px
        ddy = y - py
        ddz = z - pz
        nd = (ddx * ddx + ddy * ddy) + ddz * ddz
        return jnp.minimum(d, nd)

    jax.lax.fori_loop(1, m, body, d0)


def _fps(pts, m):
    bsz, _, n = pts.shape
    rows = n // 128
    pts4 = jnp.reshape(pts, (bsz, 3, rows, 128))
    kern = functools.partial(_fps_body, n=n, m=m, rows=rows)
    return pl.pallas_call(
        kern,
        grid=(bsz,),
        in_specs=[pl.BlockSpec((1, 3, rows, 128), lambda b: (b, 0, 0, 0))],
        out_specs=[
            pl.BlockSpec((1, m, 1), lambda b: (b, 0, 0)),
            pl.BlockSpec((1, m, 3), lambda b: (b, 0, 0)),
        ],
        out_shape=[
            jax.ShapeDtypeStruct((bsz, m, 1), jnp.int32),
            jax.ShapeDtypeStruct((bsz, m, 3), jnp.float32),
        ],
    )(pts4)


def _edge_body(*refs, mode, nq, nk, cin, cout, tq, ck):
    if mode == "l1":
        (pqt_ref, pk_ref, stemw_ref, stemb_ref, wt_ref, g_ref, b_ref,
         out_ref, dscr, fkscr, amax_scr, amin_scr) = refs
    elif mode == "gather":
        (pqt_ref, pk_ref, fk_ref, qidx_ref, wt_ref, g_ref, b_ref,
         out_ref, dscr, fkscr, amax_scr, amin_scr) = refs
    else:
        (pqt_ref, pk_ref, fk_ref, fq_ref, wt_ref, g_ref, b_ref,
         out_ref, dscr, fkscr, amax_scr, amin_scr) = refs

    nck = nk // ck
    ntiles = nq // tq
    cg = cout // _GROUPS
    bf = jnp.bfloat16
    w1b = wt_ref[0:cin, :].astype(bf)
    w2b = wt_ref[cin:2 * cin, :].astype(bf)
    if mode == "l1":
        swb = stemw_ref[...].astype(bf)
        sb = stemb_ref[0:1, :]

    def _stem(pts_t):
        return jnp.dot(pts_t.astype(bf), swb,
                       preferred_element_type=jnp.float32) + sb

    def _fk_chunk(off):
        if mode == "l1":
            return fkscr[pl.ds(off, ck), :]
        return fk_ref[0, pl.ds(off, ck), :]

    if mode == "l1":
        for c in range(nck):
            off = c * ck
            fkscr[pl.ds(off, ck), :] = _stem(pqt_ref[0, pl.ds(off, ck), :])

    def tile_body(t, carry):
        s_sum, s_sq = carry
        qoff = t * tq
        pqt = pqt_ref[0, pl.ds(qoff, tq), :]
        qx = pqt[:, 0:1]
        qy = pqt[:, 1:2]
        qz = pqt[:, 2:3]
        kq = (qx * qx + qy * qy) + qz * qz
        pqtb = pqt.astype(bf)

        if mode == "l1":
            fq = _stem(pqt)
        elif mode == "gather":
            qid = qidx_ref[0, pl.ds(qoff, tq), :]
            fq = jnp.zeros((tq, cin), jnp.float32)
            for c in range(nck):
                off = c * ck
                iota = jax.lax.broadcasted_iota(jnp.int32, (tq, ck), 1) + off
                oh = (iota == qid).astype(jnp.float32)
                fq = fq + jnp.dot(oh, fk_ref[0, pl.ds(off, ck), :],
                                  preferred_element_type=jnp.float32)
        else:
            fq = fq_ref[0, pl.ds(qoff, tq), :]
        hq = jnp.dot(fq.astype(bf), w2b, preferred_element_type=jnp.float32)

        rmin = jnp.full((tq, 1), _INF, jnp.float32)
        rarg = jnp.zeros((tq, 1), jnp.int32)
        for c in range(nck):
            off = c * ck
            kx = pk_ref[0, 0:1, pl.ds(off, ck)]
            ky = pk_ref[0, 1:2, pl.ds(off, ck)]
            kz = pk_ref[0, 2:3, pl.ds(off, ck)]
            pkc = pk_ref[0, :, pl.ds(off, ck)].astype(bf)
            cross = jnp.dot(pqtb, pkc, preferred_element_type=jnp.float32)
            kk = (kx * kx + ky * ky) + kz * kz
            dch = (kq - 2.0 * cross) + kk
            dscr[:, pl.ds(off, ck)] = dch
            cmin = jnp.min(dch, axis=1, keepdims=True)
            iota = jax.lax.broadcasted_iota(jnp.int32, (tq, ck), 1) + off
            carg = jnp.min(jnp.where(dch == cmin, iota, nk),
                           axis=1, keepdims=True)
            take = cmin < rmin
            rarg = jnp.where(take, carg, rarg)
            rmin = jnp.where(take, cmin, rmin)

        def step(_, sc):
            rarg, s1, s2, mx, mn = sc
            grow = jnp.zeros((tq, cin), jnp.float32)
            for c in range(nck):
                off = c * ck
                dch = dscr[:, pl.ds(off, ck)]
                iota = jax.lax.broadcasted_iota(jnp.int32, (tq, ck), 1) + off
                eq = iota == rarg
                grow = grow + jnp.dot(eq.astype(jnp.float32), _fk_chunk(off),
                                      preferred_element_type=jnp.float32)
                dscr[:, pl.ds(off, ck)] = jnp.where(eq, _INF, dch)
            nrmin = jnp.full((tq, 1), _INF, jnp.float32)
            nrarg = jnp.zeros((tq, 1), jnp.int32)
            for c in range(nck):
                off = c * ck
                dch = dscr[:, pl.ds(off, ck)]
                cmin = jnp.min(dch, axis=1, keepdims=True)
                iota = jax.lax.broadcasted_iota(jnp.int32, (tq, ck), 1) + off
                carg = jnp.min(jnp.where(dch == cmin, iota, nk),
                               axis=1, keepdims=True)
                take = cmin < nrmin
                nrarg = jnp.where(take, carg, nrarg)
                nrmin = jnp.where(take, cmin, nrmin)
            diff = (grow - fq).astype(bf)
            out_t = jnp.dot(diff, w1b,
                            preferred_element_type=jnp.float32) + hq
            return (nrarg, s1 + out_t, s2 + out_t * out_t,
                    jnp.maximum(mx, out_t), jnp.minimum(mn, out_t))

        init = (rarg,
                jnp.zeros((tq, cout), jnp.float32),
                jnp.zeros((tq, cout), jnp.float32),
                jnp.full((tq, cout), -_INF, jnp.float32),
                jnp.full((tq, cout), _INF, jnp.float32))
        _, s1, s2, mx, mn = jax.lax.fori_loop(0, _K, step, init)
        amax_scr[pl.ds(qoff, tq), :] = mx
        amin_scr[pl.ds(qoff, tq), :] = mn
        ts = jnp.sum(s1, axis=0, keepdims=True)
        tq_sq = jnp.sum(s2, axis=0, keepdims=True)
        return (s_sum + ts, s_sq + tq_sq)

    zc = jnp.zeros((1, cout), jnp.float32)
    s_sum, s_sq = jax.lax.fori_loop(0, ntiles, tile_body, (zc, zc))

    gmask = (jax.lax.broadcasted_iota(jnp.int32, (cout, _GROUPS), 0) // cg
             == jax.lax.broadcasted_iota(jnp.int32, (cout, _GROUPS), 1)
             ).astype(jnp.float32)
    gmask2 = (jax.lax.broadcasted_iota(jnp.int32, (_GROUPS, cout), 0)
              == jax.lax.broadcasted_iota(jnp.int32, (_GROUPS, cout), 1) // cg
              ).astype(jnp.float32)
    cnt = float(nq * _K * cg)
    sg = jnp.dot(s_sum, gmask, preferred_element_type=jnp.float32) / cnt
    qg = jnp.dot(s_sq, gmask, preferred_element_type=jnp.float32) / cnt
    vg = qg - sg * sg
    mean_c = jnp.dot(sg, gmask2, preferred_element_type=jnp.float32)
    var_c = jnp.dot(vg, gmask2, preferred_element_type=jnp.float32)
    denom = jnp.sqrt(var_c + _EPS)
    gam = g_ref[0:1, :]
    bet = b_ref[0:1, :]

    for t in range(ntiles):
        qoff = t * tq
        a_hi = amax_scr[pl.ds(qoff, tq), :]
        a_lo = amin_scr[pl.ds(qoff, tq), :]
        sel = jnp.where(gam >= 0.0, a_hi, a_lo)
        yv = (sel - mean_c) / denom
        yv = yv * gam + bet
        out_ref[0, pl.ds(qoff, tq), :] = jnp.maximum(yv, 0.0)


def _edge_layer(pqt, pk, wt, g, b, *, mode, fk=None, qidx=None, fq=None,
                stemw=None, stemb=None, tq=128, ck=256):
    bsz, nq, _ = pqt.shape
    nk = pk.shape[2]
    cin = wt.shape[0] // 2
    cout = wt.shape[1]
    tq = min(tq, nq)
    ck = min(ck, nk)
    kern = functools.partial(_edge_body, mode=mode, nq=nq, nk=nk,
                             cin=cin, cout=cout, tq=tq, ck=ck)
    full2 = lambda b_: (0, 0)
    operands = [pqt, pk]
    in_specs = [pl.BlockSpec((1, nq, 3), lambda b_: (b_, 0, 0)),
                pl.BlockSpec((1, 3, nk), lambda b_: (b_, 0, 0))]
    if mode == "l1":
        operands += [stemw, stemb]
        in_specs += [pl.BlockSpec((3, stemw.shape[1]), full2),
                     pl.BlockSpec((1, stemb.shape[1]), full2)]
    elif mode == "gather":
        operands += [fk, qidx]
        in_specs += [pl.BlockSpec((1, nk, cin), lambda b_: (b_, 0, 0)),
                     pl.BlockSpec((1, nq, 1), lambda b_: (b_, 0, 0))]
    else:
        operands += [fk, fq]
        in_specs += [pl.BlockSpec((1, nk, cin), lambda b_: (b_, 0, 0)),
                     pl.BlockSpec((1, nq, cin), lambda b_: (b_, 0, 0))]
    operands += [wt, g, b]
    in_specs += [pl.BlockSpec((2 * cin, cout), full2),
                 pl.BlockSpec((1, cout), full2),
                 pl.BlockSpec((1, cout), full2)]
    return pl.pallas_call(
        kern,
        grid=(bsz,),
        in_specs=in_specs,
        out_specs=pl.BlockSpec((1, nq, cout), lambda b_: (b_, 0, 0)),
        out_shape=jax.ShapeDtypeStruct((bsz, nq, cout), jnp.float32),
        scratch_shapes=[
            pltpu.VMEM((tq, nk), jnp.float32),
            pltpu.VMEM((nk, cin), jnp.float32),
            pltpu.VMEM((nq, cout), jnp.float32),
            pltpu.VMEM((nq, cout), jnp.float32),
        ],
    )(*operands)


def kernel(xyz, stem_W, stem_b, l1_W, l1_g, l1_b, l2_W, l2_g, l2_b,
           l3_W, l3_g, l3_b, l4_W, l4_g, l4_b):
    bsz, _, n0 = xyz.shape
    n1 = max(1, int(n0 * 0.25))
    n2 = max(1, int(n1 * 0.25))

    p0 = xyz
    p0t = jnp.transpose(xyz, (0, 2, 1))
    stemw = jnp.transpose(stem_W)
    stemb = jnp.reshape(stem_b, (1, -1))
    row = lambda v: jnp.reshape(v, (1, -1))

    f0a = _edge_layer(p0t, p0, jnp.transpose(l1_W), row(l1_g), row(l1_b),
                      mode="l1", stemw=stemw, stemb=stemb)

    idx1, p1t = _fps(p0, n1)
    p1 = jnp.transpose(p1t, (0, 2, 1))

    f1a = _edge_layer(p1t, p0, jnp.transpose(l2_W), row(l2_g), row(l2_b),
                      mode="gather", fk=f0a, qidx=idx1)

    idx2, p2t = _fps(p1, n2)

    f2_mid = _edge_layer(p2t, p1, jnp.transpose(l3_W), row(l3_g), row(l3_b),
                         mode="gather", fk=f1a, qidx=idx2)
    f2a = _edge_layer(p2t, p1, jnp.transpose(l4_W), row(l4_g), row(l4_b),
                      mode="direct", fk=f1a, fq=f2_mid)
    return jnp.transpose(f2a, (0, 2, 1))


# speedup vs baseline: 1.3736x; 1.3736x over previous
import functools
from typing import Optional

import jax
import jax.numpy as jnp
from jax.experimental import pallas as pl
from jax.experimental.pallas import tpu as pltpu

_K = 16
_GROUPS = 8
_EPS = 1e-5
_INF = float("inf")


def _fps_body(pts_ref, idx_ref, pt_ref, *, n, m, rows):
    x = pts_ref[0, 0]
    y = pts_ref[0, 1]
    z = pts_ref[0, 2]
    fi = (jax.lax.broadcasted_iota(jnp.int32, (rows, 128), 0) * 128
          + jax.lax.broadcasted_iota(jnp.int32, (rows, 128), 1))
    px0 = x[0, 0]
    py0 = y[0, 0]
    pz0 = z[0, 0]
    dx = x - px0
    dy = y - py0
    dz = z - pz0
    d0 = (dx * dx + dy * dy) + dz * dz
    idx_ref[0, 0:1, 0:1] = jnp.zeros((1, 1), jnp.int32)
    pt_ref[0, 0:1, 0:1] = jnp.reshape(px0, (1, 1))
    pt_ref[0, 0:1, 1:2] = jnp.reshape(py0, (1, 1))
    pt_ref[0, 0:1, 2:3] = jnp.reshape(pz0, (1, 1))

    def body(i, d):
        rmax = jnp.max(d)
        nxt = jnp.min(jnp.where(d == rmax, fi, n))
        sel = fi == nxt
        px = jnp.sum(jnp.where(sel, x, 0.0))
        py = jnp.sum(jnp.where(sel, y, 0.0))
        pz = jnp.sum(jnp.where(sel, z, 0.0))
        idx_ref[0, pl.ds(i, 1), 0:1] = jnp.reshape(nxt, (1, 1))
        pt_ref[0, pl.ds(i, 1), 0:1] = jnp.reshape(px, (1, 1))
        pt_ref[0, pl.ds(i, 1), 1:2] = jnp.reshape(py, (1, 1))
        pt_ref[0, pl.ds(i, 1), 2:3] = jnp.reshape(pz, (1, 1))
        ddx = x - px
        ddy = y - py
        ddz = z - pz
        nd = (ddx * ddx + ddy * ddy) + ddz * ddz
        return jnp.minimum(d, nd)

    jax.lax.fori_loop(1, m, body, d0)


def _fps(pts, m):
    bsz, _, n = pts.shape
    rows = n // 128
    pts4 = jnp.reshape(pts, (bsz, 3, rows, 128))
    kern = functools.partial(_fps_body, n=n, m=m, rows=rows)
    return pl.pallas_call(
        kern,
        grid=(bsz,),
        in_specs=[pl.BlockSpec((1, 3, rows, 128), lambda b: (b, 0, 0, 0))],
        out_specs=[
            pl.BlockSpec((1, m, 1), lambda b: (b, 0, 0)),
            pl.BlockSpec((1, m, 3), lambda b: (b, 0, 0)),
        ],
        out_shape=[
            jax.ShapeDtypeStruct((bsz, m, 1), jnp.int32),
            jax.ShapeDtypeStruct((bsz, m, 3), jnp.float32),
        ],
    )(pts4)


def _edge_body(*refs, mode, nq, nk, cin, cout, tq, ck):
    if mode == "l1":
        (pqt_ref, pk_ref, stemw_ref, stemb_ref, wt_ref, g_ref, b_ref,
         out_ref, dscr, fkscr, amax_scr, amin_scr) = refs
    elif mode == "gather":
        (pqt_ref, pk_ref, fk_ref, qidx_ref, wt_ref, g_ref, b_ref,
         out_ref, dscr, fkscr, amax_scr, amin_scr) = refs
    else:
        (pqt_ref, pk_ref, fk_ref, fq_ref, wt_ref, g_ref, b_ref,
         out_ref, dscr, fkscr, amax_scr, amin_scr) = refs

    nck = nk // ck
    ntiles = nq // tq
    cg = cout // _GROUPS
    bf = jnp.bfloat16
    w1b = wt_ref[0:cin, :].astype(bf)
    w2b = wt_ref[cin:2 * cin, :].astype(bf)
    if mode == "l1":
        swb = stemw_ref[...].astype(bf)
        sb = stemb_ref[0:1, :]

    def _stem(pts_t):
        return jnp.dot(pts_t.astype(bf), swb,
                       preferred_element_type=jnp.float32) + sb

    def _fk_chunk(off):
        if mode == "l1":
            return fkscr[pl.ds(off, ck), :]
        return fk_ref[0, pl.ds(off, ck), :]

    if mode == "l1":
        for c in range(nck):
            off = c * ck
            fkscr[pl.ds(off, ck), :] = _stem(pqt_ref[0, pl.ds(off, ck), :])

    def tile_body(t, carry):
        s_sum, s_sq = carry
        qoff = t * tq
        pqt = pqt_ref[0, pl.ds(qoff, tq), :]
        qx = pqt[:, 0:1]
        qy = pqt[:, 1:2]
        qz = pqt[:, 2:3]
        kq = (qx * qx + qy * qy) + qz * qz
        pqtb = pqt.astype(bf)

        if mode == "l1":
            fq = _stem(pqt)
        elif mode == "gather":
            qid = qidx_ref[0, pl.ds(qoff, tq), :]
            fq = jnp.zeros((tq, cin), jnp.float32)
            for c in range(nck):
                off = c * ck
                iota = jax.lax.broadcasted_iota(jnp.int32, (tq, ck), 1) + off
                oh = (iota == qid).astype(jnp.float32)
                fq = fq + jnp.dot(oh, fk_ref[0, pl.ds(off, ck), :],
                                  preferred_element_type=jnp.float32)
        else:
            fq = fq_ref[0, pl.ds(qoff, tq), :]
        hq = jnp.dot(fq.astype(bf), w2b, preferred_element_type=jnp.float32)

        for c in range(nck):
            off = c * ck
            kx = pk_ref[0, 0:1, pl.ds(off, ck)]
            ky = pk_ref[0, 1:2, pl.ds(off, ck)]
            kz = pk_ref[0, 2:3, pl.ds(off, ck)]
            pkc = pk_ref[0, :, pl.ds(off, ck)].astype(bf)
            cross = jnp.dot(pqtb, pkc, preferred_element_type=jnp.float32)
            kk = (kx * kx + ky * ky) + kz * kz
            dscr[:, pl.ds(off, ck)] = (kq - 2.0 * cross) + kk

        def step(_, sc):
            s1, s2, mx, mn = sc
            rmin = jnp.full((tq, 1), _INF, jnp.float32)
            rarg = jnp.zeros((tq, 1), jnp.int32)
            for c in range(nck):
                off = c * ck
                dch = dscr[:, pl.ds(off, ck)]
                cmin = jnp.min(dch, axis=1, keepdims=True)
                iota = jax.lax.broadcasted_iota(jnp.int32, (tq, ck), 1) + off
                carg = jnp.min(jnp.where(dch == cmin, iota, nk),
                               axis=1, keepdims=True)
                take = cmin < rmin
                rarg = jnp.where(take, carg, rarg)
                rmin = jnp.where(take, cmin, rmin)
            grow = jnp.zeros((tq, cin), jnp.float32)
            for c in range(nck):
                off = c * ck
                dch = dscr[:, pl.ds(off, ck)]
                iota = jax.lax.broadcasted_iota(jnp.int32, (tq, ck), 1) + off
                eq = iota == rarg
                grow = grow + jnp.dot(eq.astype(jnp.float32), _fk_chunk(off),
                                      preferred_element_type=jnp.float32)
                dscr[:, pl.ds(off, ck)] = jnp.where(eq, _INF, dch)
            diff = (grow - fq).astype(bf)
            out_t = jnp.dot(diff, w1b,
                            preferred_element_type=jnp.float32) + hq
            return (s1 + out_t, s2 + out_t * out_t,
                    jnp.maximum(mx, out_t), jnp.minimum(mn, out_t))

        init = (jnp.zeros((tq, cout), jnp.float32),
                jnp.zeros((tq, cout), jnp.float32),
                jnp.full((tq, cout), -_INF, jnp.float32),
                jnp.full((tq, cout), _INF, jnp.float32))
        s1, s2, mx, mn = jax.lax.fori_loop(0, _K, step, init)
        amax_scr[pl.ds(qoff, tq), :] = mx
        amin_scr[pl.ds(qoff, tq), :] = mn
        ts = jnp.sum(s1, axis=0, keepdims=True)
        tq_sq = jnp.sum(s2, axis=0, keepdims=True)
        return (s_sum + ts, s_sq + tq_sq)

    zc = jnp.zeros((1, cout), jnp.float32)
    s_sum, s_sq = jax.lax.fori_loop(0, ntiles, tile_body, (zc, zc))

    gmask = (jax.lax.broadcasted_iota(jnp.int32, (cout, _GROUPS), 0) // cg
             == jax.lax.broadcasted_iota(jnp.int32, (cout, _GROUPS), 1)
             ).astype(jnp.float32)
    gmask2 = (jax.lax.broadcasted_iota(jnp.int32, (_GROUPS, cout), 0)
              == jax.lax.broadcasted_iota(jnp.int32, (_GROUPS, cout), 1) // cg
              ).astype(jnp.float32)
    cnt = float(nq * _K * cg)
    sg = jnp.dot(s_sum, gmask, preferred_element_type=jnp.float32) / cnt
    qg = jnp.dot(s_sq, gmask, preferred_element_type=jnp.float32) / cnt
    vg = qg - sg * sg
    mean_c = jnp.dot(sg, gmask2, preferred_element_type=jnp.float32)
    var_c = jnp.dot(vg, gmask2, preferred_element_type=jnp.float32)
    denom = jnp.sqrt(var_c + _EPS)
    gam = g_ref[0:1, :]
    bet = b_ref[0:1, :]

    for t in range(ntiles):
        qoff = t * tq
        a_hi = amax_scr[pl.ds(qoff, tq), :]
        a_lo = amin_scr[pl.ds(qoff, tq), :]
        sel = jnp.where(gam >= 0.0, a_hi, a_lo)
        yv = (sel - mean_c) / denom
        yv = yv * gam + bet
        out_ref[0, pl.ds(qoff, tq), :] = jnp.maximum(yv, 0.0)


def _edge_layer(pqt, pk, wt, g, b, *, mode, fk=None, qidx=None, fq=None,
                stemw=None, stemb=None, tq=128, ck=256):
    bsz, nq, _ = pqt.shape
    nk = pk.shape[2]
    cin = wt.shape[0] // 2
    cout = wt.shape[1]
    tq = min(tq, nq)
    ck = min(ck, nk)
    kern = functools.partial(_edge_body, mode=mode, nq=nq, nk=nk,
                             cin=cin, cout=cout, tq=tq, ck=ck)
    full2 = lambda b_: (0, 0)
    operands = [pqt, pk]
    in_specs = [pl.BlockSpec((1, nq, 3), lambda b_: (b_, 0, 0)),
                pl.BlockSpec((1, 3, nk), lambda b_: (b_, 0, 0))]
    if mode == "l1":
        operands += [stemw, stemb]
        in_specs += [pl.BlockSpec((3, stemw.shape[1]), full2),
                     pl.BlockSpec((1, stemb.shape[1]), full2)]
    elif mode == "gather":
        operands += [fk, qidx]
        in_specs += [pl.BlockSpec((1, nk, cin), lambda b_: (b_, 0, 0)),
                     pl.BlockSpec((1, nq, 1), lambda b_: (b_, 0, 0))]
    else:
        operands += [fk, fq]
        in_specs += [pl.BlockSpec((1, nk, cin), lambda b_: (b_, 0, 0)),
                     pl.BlockSpec((1, nq, cin), lambda b_: (b_, 0, 0))]
    operands += [wt, g, b]
    in_specs += [pl.BlockSpec((2 * cin, cout), full2),
                 pl.BlockSpec((1, cout), full2),
                 pl.BlockSpec((1, cout), full2)]
    return pl.pallas_call(
        kern,
        grid=(bsz,),
        in_specs=in_specs,
        out_specs=pl.BlockSpec((1, nq, cout), lambda b_: (b_, 0, 0)),
        out_shape=jax.ShapeDtypeStruct((bsz, nq, cout), jnp.float32),
        scratch_shapes=[
            pltpu.VMEM((tq, nk), jnp.float32),
            pltpu.VMEM((nk, cin), jnp.float32),
            pltpu.VMEM((nq, cout), jnp.float32),
            pltpu.VMEM((nq, cout), jnp.float32),
        ],
    )(*operands)


def kernel(xyz, stem_W, stem_b, l1_W, l1_g, l1_b, l2_W, l2_g, l2_b,
           l3_W, l3_g, l3_b, l4_W, l4_g, l4_b):
    bsz, _, n0 = xyz.shape
    n1 = max(1, int(n0 * 0.25))
    n2 = max(1, int(n1 * 0.25))

    p0 = xyz
    p0t = jnp.transpose(xyz, (0, 2, 1))
    stemw = jnp.transpose(stem_W)
    stemb = jnp.reshape(stem_b, (1, -1))
    row = lambda v: jnp.reshape(v, (1, -1))

    f0a = _edge_layer(p0t, p0, jnp.transpose(l1_W), row(l1_g), row(l1_b),
                      mode="l1", stemw=stemw, stemb=stemb)

    idx1, p1t = _fps(p0, n1)
    p1 = jnp.transpose(p1t, (0, 2, 1))

    f1a = _edge_layer(p1t, p0, jnp.transpose(l2_W), row(l2_g), row(l2_b),
                      mode="gather", fk=f0a, qidx=idx1)

    idx2, p2t = _fps(p1, n2)

    f2_mid = _edge_layer(p2t, p1, jnp.transpose(l3_W), row(l3_g), row(l3_b),
                         mode="gather", fk=f1a, qidx=idx2)
    f2a = _edge_layer(p2t, p1, jnp.transpose(l4_W), row(l4_g), row(l4_b),
                      mode="direct", fk=f1a, fq=f2_mid)
    return jnp.transpose(f2a, (0, 2, 1))


# speedup vs baseline: 1.7391x; 1.2661x over previous
import functools
from typing import Optional

import jax
import jax.numpy as jnp
from jax.experimental import pallas as pl
from jax.experimental.pallas import tpu as pltpu

_K = 16
_GROUPS = 8
_EPS = 1e-5
_INF = float("inf")


def _fps_body(pts_ref, idx_ref, pt_ref, *, n, m, rows):
    x = pts_ref[0, 0]
    y = pts_ref[0, 1]
    z = pts_ref[0, 2]
    fi = (jax.lax.broadcasted_iota(jnp.int32, (rows, 128), 0) * 128
          + jax.lax.broadcasted_iota(jnp.int32, (rows, 128), 1))
    px0 = x[0, 0]
    py0 = y[0, 0]
    pz0 = z[0, 0]
    dx = x - px0
    dy = y - py0
    dz = z - pz0
    d0 = (dx * dx + dy * dy) + dz * dz
    idx_ref[0, 0:1, 0:1] = jnp.zeros((1, 1), jnp.int32)
    pt_ref[0, 0:1, 0:1] = jnp.reshape(px0, (1, 1))
    pt_ref[0, 0:1, 1:2] = jnp.reshape(py0, (1, 1))
    pt_ref[0, 0:1, 2:3] = jnp.reshape(pz0, (1, 1))

    def body(i, d):
        rmax = jnp.max(d)
        nxt = jnp.min(jnp.where(d == rmax, fi, n))
        sel = fi == nxt
        px = jnp.sum(jnp.where(sel, x, 0.0))
        py = jnp.sum(jnp.where(sel, y, 0.0))
        pz = jnp.sum(jnp.where(sel, z, 0.0))
        idx_ref[0, pl.ds(i, 1), 0:1] = jnp.reshape(nxt, (1, 1))
        pt_ref[0, pl.ds(i, 1), 0:1] = jnp.reshape(px, (1, 1))
        pt_ref[0, pl.ds(i, 1), 1:2] = jnp.reshape(py, (1, 1))
        pt_ref[0, pl.ds(i, 1), 2:3] = jnp.reshape(pz, (1, 1))
        ddx = x - px
        ddy = y - py
        ddz = z - pz
        nd = (ddx * ddx + ddy * ddy) + ddz * ddz
        return jnp.minimum(d, nd)

    jax.lax.fori_loop(1, m, body, d0)


def _fps(pts, m):
    bsz, _, n = pts.shape
    rows = n // 128
    pts4 = jnp.reshape(pts, (bsz, 3, rows, 128))
    kern = functools.partial(_fps_body, n=n, m=m, rows=rows)
    return pl.pallas_call(
        kern,
        grid=(bsz,),
        in_specs=[pl.BlockSpec((1, 3, rows, 128), lambda b: (b, 0, 0, 0))],
        out_specs=[
            pl.BlockSpec((1, m, 1), lambda b: (b, 0, 0)),
            pl.BlockSpec((1, m, 3), lambda b: (b, 0, 0)),
        ],
        out_shape=[
            jax.ShapeDtypeStruct((bsz, m, 1), jnp.int32),
            jax.ShapeDtypeStruct((bsz, m, 3), jnp.float32),
        ],
    )(pts4)


def _edge_body(*refs, mode, nq, nk, cin, cout, tq, ck):
    if mode == "l1":
        (pqt_ref, pk_ref, stemw_ref, stemb_ref, wt_ref, g_ref, b_ref,
         out_ref, dscr, fkscr, amax_scr, amin_scr) = refs
    elif mode == "gather":
        (pqt_ref, pk_ref, fk_ref, qidx_ref, wt_ref, g_ref, b_ref,
         out_ref, dscr, fkscr, amax_scr, amin_scr) = refs
    else:
        (pqt_ref, pk_ref, fk_ref, fq_ref, wt_ref, g_ref, b_ref,
         out_ref, dscr, fkscr, amax_scr, amin_scr) = refs

    nck = nk // ck
    ntiles = nq // tq
    cg = cout // _GROUPS
    bf = jnp.bfloat16
    w1b = wt_ref[0:cin, :].astype(bf)
    w2b = wt_ref[cin:2 * cin, :].astype(bf)
    if mode == "l1":
        swb = stemw_ref[...].astype(bf)
        sb = stemb_ref[0:1, :]

    def _stem(pts_t):
        return jnp.dot(pts_t.astype(bf), swb,
                       preferred_element_type=jnp.float32) + sb

    def _fk_chunk(off):
        if mode == "l1":
            return fkscr[pl.ds(off, ck), :]
        return fk_ref[0, pl.ds(off, ck), :]

    if mode == "l1":
        for c in range(nck):
            off = c * ck
            fkscr[pl.ds(off, ck), :] = _stem(pqt_ref[0, pl.ds(off, ck), :])

    def tile_body(t, carry):
        s_sum, s_sq = carry
        qoff = t * tq
        pqt = pqt_ref[0, pl.ds(qoff, tq), :]
        qx = pqt[:, 0:1]
        qy = pqt[:, 1:2]
        qz = pqt[:, 2:3]
        kq = (qx * qx + qy * qy) + qz * qz
        pqtb = pqt.astype(bf)

        if mode == "l1":
            fq = _stem(pqt)
        elif mode == "gather":
            qid = qidx_ref[0, pl.ds(qoff, tq), :]
            fq = jnp.zeros((tq, cin), jnp.float32)
            for c in range(nck):
                off = c * ck
                iota = jax.lax.broadcasted_iota(jnp.int32, (tq, ck), 1) + off
                oh = (iota == qid).astype(jnp.float32)
                fq = fq + jnp.dot(oh, fk_ref[0, pl.ds(off, ck), :],
                                  preferred_element_type=jnp.float32)
        else:
            fq = fq_ref[0, pl.ds(qoff, tq), :]
        hq = jnp.dot(fq.astype(bf), w2b, preferred_element_type=jnp.float32)

        for c in range(nck):
            off = c * ck
            kx = pk_ref[0, 0:1, pl.ds(off, ck)]
            ky = pk_ref[0, 1:2, pl.ds(off, ck)]
            kz = pk_ref[0, 2:3, pl.ds(off, ck)]
            pkc = pk_ref[0, :, pl.ds(off, ck)].astype(bf)
            cross = jnp.dot(pqtb, pkc, preferred_element_type=jnp.float32)
            kk = (kx * kx + ky * ky) + kz * kz
            dscr[:, pl.ds(off, ck)] = (kq - 2.0 * cross) + kk

        def step(_, sc):
            s1, s2, mx, mn = sc
            rmin = jnp.full((tq, 1), _INF, jnp.float32)
            rarg = jnp.zeros((tq, 1), jnp.int32)
            for c in range(nck):
                off = c * ck
                dch = dscr[:, pl.ds(off, ck)]
                cmin = jnp.min(dch, axis=1, keepdims=True)
                iota = jax.lax.broadcasted_iota(jnp.int32, (tq, ck), 1) + off
                carg = jnp.min(jnp.where(dch == cmin, iota, nk),
                               axis=1, keepdims=True)
                take = cmin < rmin
                rarg = jnp.where(take, carg, rarg)
                rmin = jnp.where(take, cmin, rmin)
            grow = jnp.zeros((tq, cin), jnp.float32)
            for c in range(nck):
                off = c * ck
                dch = dscr[:, pl.ds(off, ck)]
                iota = jax.lax.broadcasted_iota(jnp.int32, (tq, ck), 1) + off
                eq = iota == rarg
                grow = grow + jnp.dot(eq.astype(jnp.float32), _fk_chunk(off),
                                      preferred_element_type=jnp.float32)
                dscr[:, pl.ds(off, ck)] = jnp.where(eq, _INF, dch)
            diff = (grow - fq).astype(bf)
            out_t = jnp.dot(diff, w1b,
                            preferred_element_type=jnp.float32) + hq
            return (s1 + out_t, s2 + out_t * out_t,
                    jnp.maximum(mx, out_t), jnp.minimum(mn, out_t))

        init = (jnp.zeros((tq, cout), jnp.float32),
                jnp.zeros((tq, cout), jnp.float32),
                jnp.full((tq, cout), -_INF, jnp.float32),
                jnp.full((tq, cout), _INF, jnp.float32))
        s1, s2, mx, mn = jax.lax.fori_loop(0, _K, step, init)
        amax_scr[pl.ds(qoff, tq), :] = mx
        amin_scr[pl.ds(qoff, tq), :] = mn
        ts = jnp.sum(s1, axis=0, keepdims=True)
        tq_sq = jnp.sum(s2, axis=0, keepdims=True)
        return (s_sum + ts, s_sq + tq_sq)

    zc = jnp.zeros((1, cout), jnp.float32)
    s_sum, s_sq = jax.lax.fori_loop(0, ntiles, tile_body, (zc, zc))

    gmask = (jax.lax.broadcasted_iota(jnp.int32, (cout, _GROUPS), 0) // cg
             == jax.lax.broadcasted_iota(jnp.int32, (cout, _GROUPS), 1)
             ).astype(jnp.float32)
    gmask2 = (jax.lax.broadcasted_iota(jnp.int32, (_GROUPS, cout), 0)
              == jax.lax.broadcasted_iota(jnp.int32, (_GROUPS, cout), 1) // cg
              ).astype(jnp.float32)
    cnt = float(nq * _K * cg)
    sg = jnp.dot(s_sum, gmask, preferred_element_type=jnp.float32) / cnt
    qg = jnp.dot(s_sq, gmask, preferred_element_type=jnp.float32) / cnt
    vg = qg - sg * sg
    mean_c = jnp.dot(sg, gmask2, preferred_element_type=jnp.float32)
    var_c = jnp.dot(vg, gmask2, preferred_element_type=jnp.float32)
    denom = jnp.sqrt(var_c + _EPS)
    gam = g_ref[0:1, :]
    bet = b_ref[0:1, :]

    for t in range(ntiles):
        qoff = t * tq
        a_hi = amax_scr[pl.ds(qoff, tq), :]
        a_lo = amin_scr[pl.ds(qoff, tq), :]
        sel = jnp.where(gam >= 0.0, a_hi, a_lo)
        yv = (sel - mean_c) / denom
        yv = yv * gam + bet
        out_ref[0, pl.ds(qoff, tq), :] = jnp.maximum(yv, 0.0)


def _edge_layer(pqt, pk, wt, g, b, *, mode, fk=None, qidx=None, fq=None,
                stemw=None, stemb=None, tq=128, ck=512):
    bsz, nq, _ = pqt.shape
    nk = pk.shape[2]
    cin = wt.shape[0] // 2
    cout = wt.shape[1]
    tq = min(tq, nq)
    ck = min(ck, nk)
    kern = functools.partial(_edge_body, mode=mode, nq=nq, nk=nk,
                             cin=cin, cout=cout, tq=tq, ck=ck)
    full2 = lambda b_: (0, 0)
    operands = [pqt, pk]
    in_specs = [pl.BlockSpec((1, nq, 3), lambda b_: (b_, 0, 0)),
                pl.BlockSpec((1, 3, nk), lambda b_: (b_, 0, 0))]
    if mode == "l1":
        operands += [stemw, stemb]
        in_specs += [pl.BlockSpec((3, stemw.shape[1]), full2),
                     pl.BlockSpec((1, stemb.shape[1]), full2)]
    elif mode == "gather":
        operands += [fk, qidx]
        in_specs += [pl.BlockSpec((1, nk, cin), lambda b_: (b_, 0, 0)),
                     pl.BlockSpec((1, nq, 1), lambda b_: (b_, 0, 0))]
    else:
        operands += [fk, fq]
        in_specs += [pl.BlockSpec((1, nk, cin), lambda b_: (b_, 0, 0)),
                     pl.BlockSpec((1, nq, cin), lambda b_: (b_, 0, 0))]
    operands += [wt, g, b]
    in_specs += [pl.BlockSpec((2 * cin, cout), full2),
                 pl.BlockSpec((1, cout), full2),
                 pl.BlockSpec((1, cout), full2)]
    return pl.pallas_call(
        kern,
        grid=(bsz,),
        in_specs=in_specs,
        out_specs=pl.BlockSpec((1, nq, cout), lambda b_: (b_, 0, 0)),
        out_shape=jax.ShapeDtypeStruct((bsz, nq, cout), jnp.float32),
        scratch_shapes=[
            pltpu.VMEM((tq, nk), jnp.float32),
            pltpu.VMEM((nk, cin), jnp.float32),
            pltpu.VMEM((nq, cout), jnp.float32),
            pltpu.VMEM((nq, cout), jnp.float32),
        ],
    )(*operands)


def kernel(xyz, stem_W, stem_b, l1_W, l1_g, l1_b, l2_W, l2_g, l2_b,
           l3_W, l3_g, l3_b, l4_W, l4_g, l4_b):
    bsz, _, n0 = xyz.shape
    n1 = max(1, int(n0 * 0.25))
    n2 = max(1, int(n1 * 0.25))

    p0 = xyz
    p0t = jnp.transpose(xyz, (0, 2, 1))
    stemw = jnp.transpose(stem_W)
    stemb = jnp.reshape(stem_b, (1, -1))
    row = lambda v: jnp.reshape(v, (1, -1))

    f0a = _edge_layer(p0t, p0, jnp.transpose(l1_W), row(l1_g), row(l1_b),
                      mode="l1", stemw=stemw, stemb=stemb)

    idx1, p1t = _fps(p0, n1)
    p1 = jnp.transpose(p1t, (0, 2, 1))

    f1a = _edge_layer(p1t, p0, jnp.transpose(l2_W), row(l2_g), row(l2_b),
                      mode="gather", fk=f0a, qidx=idx1)

    idx2, p2t = _fps(p1, n2)

    f2_mid = _edge_layer(p2t, p1, jnp.transpose(l3_W), row(l3_g), row(l3_b),
                         mode="gather", fk=f1a, qidx=idx2)
    f2a = _edge_layer(p2t, p1, jnp.transpose(l4_W), row(l4_g), row(l4_b),
                      mode="direct", fk=f1a, fq=f2_mid)
    return jnp.transpose(f2a, (0, 2, 1))


# speedup vs baseline: 1.8085x; 1.0399x over previous
import functools
from typing import Optional

import jax
import jax.numpy as jnp
from jax.experimental import pallas as pl
from jax.experimental.pallas import tpu as pltpu

_K = 16
_GROUPS = 8
_EPS = 1e-5
_INF = float("inf")


def _fps_body(pts_ref, idx_ref, pt_ref, *, n, m, rows):
    x = pts_ref[0, 0]
    y = pts_ref[0, 1]
    z = pts_ref[0, 2]
    fi = (jax.lax.broadcasted_iota(jnp.int32, (rows, 128), 0) * 128
          + jax.lax.broadcasted_iota(jnp.int32, (rows, 128), 1))
    px0 = x[0, 0]
    py0 = y[0, 0]
    pz0 = z[0, 0]
    dx = x - px0
    dy = y - py0
    dz = z - pz0
    d0 = (dx * dx + dy * dy) + dz * dz
    idx_ref[0, 0:1, 0:1] = jnp.zeros((1, 1), jnp.int32)
    pt_ref[0, 0:1, 0:1] = jnp.reshape(px0, (1, 1))
    pt_ref[0, 0:1, 1:2] = jnp.reshape(py0, (1, 1))
    pt_ref[0, 0:1, 2:3] = jnp.reshape(pz0, (1, 1))

    def body(i, d):
        rmax = jnp.max(d)
        nxt = jnp.min(jnp.where(d == rmax, fi, n))
        sel = fi == nxt
        px = jnp.sum(jnp.where(sel, x, 0.0))
        py = jnp.sum(jnp.where(sel, y, 0.0))
        pz = jnp.sum(jnp.where(sel, z, 0.0))
        idx_ref[0, pl.ds(i, 1), 0:1] = jnp.reshape(nxt, (1, 1))
        pt_ref[0, pl.ds(i, 1), 0:1] = jnp.reshape(px, (1, 1))
        pt_ref[0, pl.ds(i, 1), 1:2] = jnp.reshape(py, (1, 1))
        pt_ref[0, pl.ds(i, 1), 2:3] = jnp.reshape(pz, (1, 1))
        ddx = x - px
        ddy = y - py
        ddz = z - pz
        nd = (ddx * ddx + ddy * ddy) + ddz * ddz
        return jnp.minimum(d, nd)

    jax.lax.fori_loop(1, m, body, d0)


def _fps(pts, m):
    bsz, _, n = pts.shape
    rows = n // 128
    pts4 = jnp.reshape(pts, (bsz, 3, rows, 128))
    kern = functools.partial(_fps_body, n=n, m=m, rows=rows)
    return pl.pallas_call(
        kern,
        grid=(bsz,),
        in_specs=[pl.BlockSpec((1, 3, rows, 128), lambda b: (b, 0, 0, 0))],
        out_specs=[
            pl.BlockSpec((1, m, 1), lambda b: (b, 0, 0)),
            pl.BlockSpec((1, m, 3), lambda b: (b, 0, 0)),
        ],
        out_shape=[
            jax.ShapeDtypeStruct((bsz, m, 1), jnp.int32),
            jax.ShapeDtypeStruct((bsz, m, 3), jnp.float32),
        ],
    )(pts4)


def _edge_body(*refs, mode, nq, nk, cin, cout, tq, ck):
    if mode == "l1":
        (pqt_ref, pk_ref, stemw_ref, stemb_ref, wt_ref, g_ref, b_ref,
         out_ref, dscr, fkscr, amax_scr, amin_scr) = refs
    elif mode == "gather":
        (pqt_ref, pk_ref, fk_ref, qidx_ref, wt_ref, g_ref, b_ref,
         out_ref, dscr, fkscr, amax_scr, amin_scr) = refs
    else:
        (pqt_ref, pk_ref, fk_ref, fq_ref, wt_ref, g_ref, b_ref,
         out_ref, dscr, fkscr, amax_scr, amin_scr) = refs

    nck = nk // ck
    ntiles = nq // tq
    cg = cout // _GROUPS
    bf = jnp.bfloat16
    w1b = wt_ref[0:cin, :].astype(bf)
    w2b = wt_ref[cin:2 * cin, :].astype(bf)
    if mode == "l1":
        swb = stemw_ref[...].astype(bf)
        sb = stemb_ref[0:1, :]

    def _stem(pts_t):
        return jnp.dot(pts_t.astype(bf), swb,
                       preferred_element_type=jnp.float32) + sb

    def _fk_chunk(off):
        if mode == "l1":
            return fkscr[pl.ds(off, ck), :]
        return fk_ref[0, pl.ds(off, ck), :]

    if mode == "l1":
        for c in range(nck):
            off = c * ck
            fkscr[pl.ds(off, ck), :] = _stem(pqt_ref[0, pl.ds(off, ck), :])

    def tile_body(t, carry):
        s_sum, s_sq = carry
        qoff = t * tq
        pqt = pqt_ref[0, pl.ds(qoff, tq), :]
        qx = pqt[:, 0:1]
        qy = pqt[:, 1:2]
        qz = pqt[:, 2:3]
        kq = (qx * qx + qy * qy) + qz * qz
        pqtb = pqt.astype(bf)

        if mode == "l1":
            fq = _stem(pqt)
        elif mode == "gather":
            qid = qidx_ref[0, pl.ds(qoff, tq), :]
            fq = jnp.zeros((tq, cin), jnp.float32)
            for c in range(nck):
                off = c * ck
                iota = jax.lax.broadcasted_iota(jnp.int32, (tq, ck), 1) + off
                oh = (iota == qid).astype(jnp.float32)
                fq = fq + jnp.dot(oh, fk_ref[0, pl.ds(off, ck), :],
                                  preferred_element_type=jnp.float32)
        else:
            fq = fq_ref[0, pl.ds(qoff, tq), :]
        hq = jnp.dot(fq.astype(bf), w2b, preferred_element_type=jnp.float32)

        for c in range(nck):
            off = c * ck
            kx = pk_ref[0, 0:1, pl.ds(off, ck)]
            ky = pk_ref[0, 1:2, pl.ds(off, ck)]
            kz = pk_ref[0, 2:3, pl.ds(off, ck)]
            pkc = pk_ref[0, :, pl.ds(off, ck)].astype(bf)
            cross = jnp.dot(pqtb, pkc, preferred_element_type=jnp.float32)
            kk = (kx * kx + ky * ky) + kz * kz
            dscr[:, pl.ds(off, ck)] = (kq - 2.0 * cross) + kk

        def step(_, sc):
            s1, s2, mx, mn = sc
            rmin = jnp.full((tq, 1), _INF, jnp.float32)
            rarg = jnp.zeros((tq, 1), jnp.int32)
            for c in range(nck):
                off = c * ck
                dch = dscr[:, pl.ds(off, ck)]
                cmin = jnp.min(dch, axis=1, keepdims=True)
                iota = jax.lax.broadcasted_iota(jnp.int32, (tq, ck), 1) + off
                carg = jnp.min(jnp.where(dch == cmin, iota, nk),
                               axis=1, keepdims=True)
                take = cmin < rmin
                rarg = jnp.where(take, carg, rarg)
                rmin = jnp.where(take, cmin, rmin)
            grow = jnp.zeros((tq, cin), jnp.float32)
            for c in range(nck):
                off = c * ck
                dch = dscr[:, pl.ds(off, ck)]
                iota = jax.lax.broadcasted_iota(jnp.int32, (tq, ck), 1) + off
                eq = iota == rarg
                grow = grow + jnp.dot(eq.astype(jnp.float32), _fk_chunk(off),
                                      preferred_element_type=jnp.float32)
                dscr[:, pl.ds(off, ck)] = jnp.where(eq, _INF, dch)
            diff = (grow - fq).astype(bf)
            out_t = jnp.dot(diff, w1b,
                            preferred_element_type=jnp.float32) + hq
            return (s1 + out_t, s2 + out_t * out_t,
                    jnp.maximum(mx, out_t), jnp.minimum(mn, out_t))

        init = (jnp.zeros((tq, cout), jnp.float32),
                jnp.zeros((tq, cout), jnp.float32),
                jnp.full((tq, cout), -_INF, jnp.float32),
                jnp.full((tq, cout), _INF, jnp.float32))
        s1, s2, mx, mn = jax.lax.fori_loop(0, _K, step, init)
        amax_scr[pl.ds(qoff, tq), :] = mx
        amin_scr[pl.ds(qoff, tq), :] = mn
        ts = jnp.sum(s1, axis=0, keepdims=True)
        tq_sq = jnp.sum(s2, axis=0, keepdims=True)
        return (s_sum + ts, s_sq + tq_sq)

    zc = jnp.zeros((1, cout), jnp.float32)
    s_sum, s_sq = jax.lax.fori_loop(0, ntiles, tile_body, (zc, zc))

    gmask = (jax.lax.broadcasted_iota(jnp.int32, (cout, _GROUPS), 0) // cg
             == jax.lax.broadcasted_iota(jnp.int32, (cout, _GROUPS), 1)
             ).astype(jnp.float32)
    gmask2 = (jax.lax.broadcasted_iota(jnp.int32, (_GROUPS, cout), 0)
              == jax.lax.broadcasted_iota(jnp.int32, (_GROUPS, cout), 1) // cg
              ).astype(jnp.float32)
    cnt = float(nq * _K * cg)
    sg = jnp.dot(s_sum, gmask, preferred_element_type=jnp.float32) / cnt
    qg = jnp.dot(s_sq, gmask, preferred_element_type=jnp.float32) / cnt
    vg = qg - sg * sg
    mean_c = jnp.dot(sg, gmask2, preferred_element_type=jnp.float32)
    var_c = jnp.dot(vg, gmask2, preferred_element_type=jnp.float32)
    denom = jnp.sqrt(var_c + _EPS)
    gam = g_ref[0:1, :]
    bet = b_ref[0:1, :]

    for t in range(ntiles):
        qoff = t * tq
        a_hi = amax_scr[pl.ds(qoff, tq), :]
        a_lo = amin_scr[pl.ds(qoff, tq), :]
        sel = jnp.where(gam >= 0.0, a_hi, a_lo)
        yv = (sel - mean_c) / denom
        yv = yv * gam + bet
        out_ref[0, pl.ds(qoff, tq), :] = jnp.maximum(yv, 0.0)


def _edge_layer(pqt, pk, wt, g, b, *, mode, fk=None, qidx=None, fq=None,
                stemw=None, stemb=None, tq=128, ck=1024):
    bsz, nq, _ = pqt.shape
    nk = pk.shape[2]
    cin = wt.shape[0] // 2
    cout = wt.shape[1]
    tq = min(tq, nq)
    ck = min(ck, nk)
    kern = functools.partial(_edge_body, mode=mode, nq=nq, nk=nk,
                             cin=cin, cout=cout, tq=tq, ck=ck)
    full2 = lambda b_: (0, 0)
    operands = [pqt, pk]
    in_specs = [pl.BlockSpec((1, nq, 3), lambda b_: (b_, 0, 0)),
                pl.BlockSpec((1, 3, nk), lambda b_: (b_, 0, 0))]
    if mode == "l1":
        operands += [stemw, stemb]
        in_specs += [pl.BlockSpec((3, stemw.shape[1]), full2),
                     pl.BlockSpec((1, stemb.shape[1]), full2)]
    elif mode == "gather":
        operands += [fk, qidx]
        in_specs += [pl.BlockSpec((1, nk, cin), lambda b_: (b_, 0, 0)),
                     pl.BlockSpec((1, nq, 1), lambda b_: (b_, 0, 0))]
    else:
        operands += [fk, fq]
        in_specs += [pl.BlockSpec((1, nk, cin), lambda b_: (b_, 0, 0)),
                     pl.BlockSpec((1, nq, cin), lambda b_: (b_, 0, 0))]
    operands += [wt, g, b]
    in_specs += [pl.BlockSpec((2 * cin, cout), full2),
                 pl.BlockSpec((1, cout), full2),
                 pl.BlockSpec((1, cout), full2)]
    return pl.pallas_call(
        kern,
        grid=(bsz,),
        in_specs=in_specs,
        out_specs=pl.BlockSpec((1, nq, cout), lambda b_: (b_, 0, 0)),
        out_shape=jax.ShapeDtypeStruct((bsz, nq, cout), jnp.float32),
        scratch_shapes=[
            pltpu.VMEM((tq, nk), jnp.float32),
            pltpu.VMEM((nk, cin), jnp.float32),
            pltpu.VMEM((nq, cout), jnp.float32),
            pltpu.VMEM((nq, cout), jnp.float32),
        ],
    )(*operands)


def kernel(xyz, stem_W, stem_b, l1_W, l1_g, l1_b, l2_W, l2_g, l2_b,
           l3_W, l3_g, l3_b, l4_W, l4_g, l4_b):
    bsz, _, n0 = xyz.shape
    n1 = max(1, int(n0 * 0.25))
    n2 = max(1, int(n1 * 0.25))

    p0 = xyz
    p0t = jnp.transpose(xyz, (0, 2, 1))
    stemw = jnp.transpose(stem_W)
    stemb = jnp.reshape(stem_b, (1, -1))
    row = lambda v: jnp.reshape(v, (1, -1))

    f0a = _edge_layer(p0t, p0, jnp.transpose(l1_W), row(l1_g), row(l1_b),
                      mode="l1", stemw=stemw, stemb=stemb)

    idx1, p1t = _fps(p0, n1)
    p1 = jnp.transpose(p1t, (0, 2, 1))

    f1a = _edge_layer(p1t, p0, jnp.transpose(l2_W), row(l2_g), row(l2_b),
                      mode="gather", fk=f0a, qidx=idx1)

    idx2, p2t = _fps(p1, n2)

    f2_mid = _edge_layer(p2t, p1, jnp.transpose(l3_W), row(l3_g), row(l3_b),
                         mode="gather", fk=f1a, qidx=idx2)
    f2a = _edge_layer(p2t, p1, jnp.transpose(l4_W), row(l4_g), row(l4_b),
                      mode="direct", fk=f1a, fq=f2_mid)
    return jnp.transpose(f2a, (0, 2, 1))


# speedup vs baseline: 1.9942x; 1.1027x over previous
import functools
from typing import Optional

import jax
import jax.numpy as jnp
from jax.experimental import pallas as pl
from jax.experimental.pallas import tpu as pltpu

_K = 16
_GROUPS = 8
_EPS = 1e-5
_INF = float("inf")


def _fps_body(pts_ref, idx_ref, pt_ref, *, n, m, rows):
    x = pts_ref[0, 0]
    y = pts_ref[0, 1]
    z = pts_ref[0, 2]
    fi = (jax.lax.broadcasted_iota(jnp.int32, (rows, 128), 0) * 128
          + jax.lax.broadcasted_iota(jnp.int32, (rows, 128), 1))
    px0 = x[0, 0]
    py0 = y[0, 0]
    pz0 = z[0, 0]
    dx = x - px0
    dy = y - py0
    dz = z - pz0
    d0 = (dx * dx + dy * dy) + dz * dz
    idx_ref[0, 0:1, 0:1] = jnp.zeros((1, 1), jnp.int32)
    pt_ref[0, 0:1, 0:1] = jnp.reshape(px0, (1, 1))
    pt_ref[0, 0:1, 1:2] = jnp.reshape(py0, (1, 1))
    pt_ref[0, 0:1, 2:3] = jnp.reshape(pz0, (1, 1))

    def body(i, d):
        rmax = jnp.max(d)
        nxt = jnp.min(jnp.where(d == rmax, fi, n))
        sel = fi == nxt
        px = jnp.sum(jnp.where(sel, x, 0.0))
        py = jnp.sum(jnp.where(sel, y, 0.0))
        pz = jnp.sum(jnp.where(sel, z, 0.0))
        idx_ref[0, pl.ds(i, 1), 0:1] = jnp.reshape(nxt, (1, 1))
        pt_ref[0, pl.ds(i, 1), 0:1] = jnp.reshape(px, (1, 1))
        pt_ref[0, pl.ds(i, 1), 1:2] = jnp.reshape(py, (1, 1))
        pt_ref[0, pl.ds(i, 1), 2:3] = jnp.reshape(pz, (1, 1))
        ddx = x - px
        ddy = y - py
        ddz = z - pz
        nd = (ddx * ddx + ddy * ddy) + ddz * ddz
        return jnp.minimum(d, nd)

    jax.lax.fori_loop(1, m, body, d0)


def _fps(pts, m):
    bsz, _, n = pts.shape
    rows = n // 128
    pts4 = jnp.reshape(pts, (bsz, 3, rows, 128))
    kern = functools.partial(_fps_body, n=n, m=m, rows=rows)
    return pl.pallas_call(
        kern,
        grid=(bsz,),
        in_specs=[pl.BlockSpec((1, 3, rows, 128), lambda b: (b, 0, 0, 0))],
        out_specs=[
            pl.BlockSpec((1, m, 1), lambda b: (b, 0, 0)),
            pl.BlockSpec((1, m, 3), lambda b: (b, 0, 0)),
        ],
        out_shape=[
            jax.ShapeDtypeStruct((bsz, m, 1), jnp.int32),
            jax.ShapeDtypeStruct((bsz, m, 3), jnp.float32),
        ],
    )(pts4)


def _edge_body(*refs, mode, nq, nk, cin, cout, tq, ck):
    if mode == "l1":
        (pqt_ref, pk_ref, stemw_ref, stemb_ref, wt_ref, g_ref, b_ref,
         out_ref, dscr, fkscr, amax_scr, amin_scr) = refs
    elif mode == "gather":
        (pqt_ref, pk_ref, fk_ref, qidx_ref, wt_ref, g_ref, b_ref,
         out_ref, dscr, fkscr, amax_scr, amin_scr) = refs
    else:
        (pqt_ref, pk_ref, fk_ref, fq_ref, wt_ref, g_ref, b_ref,
         out_ref, dscr, fkscr, amax_scr, amin_scr) = refs

    nck = nk // ck
    ntiles = nq // tq
    cg = cout // _GROUPS
    bf = jnp.bfloat16
    w1b = wt_ref[0:cin, :].astype(bf)
    w2b = wt_ref[cin:2 * cin, :].astype(bf)
    if mode == "l1":
        swb = stemw_ref[...].astype(bf)
        sb = stemb_ref[0:1, :]

    def _stem(pts_t):
        return jnp.dot(pts_t.astype(bf), swb,
                       preferred_element_type=jnp.float32) + sb

    def _fk_chunk(off):
        if mode == "l1":
            return fkscr[pl.ds(off, ck), :]
        return fk_ref[0, pl.ds(off, ck), :]

    if mode == "l1":
        for c in range(nck):
            off = c * ck
            fkscr[pl.ds(off, ck), :] = _stem(pqt_ref[0, pl.ds(off, ck), :])

    def tile_body(t, carry):
        s_sum, s_sq = carry
        qoff = t * tq
        pqt = pqt_ref[0, pl.ds(qoff, tq), :]
        qx = pqt[:, 0:1]
        qy = pqt[:, 1:2]
        qz = pqt[:, 2:3]
        kq = (qx * qx + qy * qy) + qz * qz
        pqtb = pqt.astype(bf)

        if mode == "l1":
            fq = _stem(pqt)
        elif mode == "gather":
            qid = qidx_ref[0, pl.ds(qoff, tq), :]
            fq = jnp.zeros((tq, cin), jnp.float32)
            for c in range(nck):
                off = c * ck
                iota = jax.lax.broadcasted_iota(jnp.int32, (tq, ck), 1) + off
                oh = (iota == qid).astype(jnp.float32)
                fq = fq + jnp.dot(oh, fk_ref[0, pl.ds(off, ck), :],
                                  preferred_element_type=jnp.float32)
        else:
            fq = fq_ref[0, pl.ds(qoff, tq), :]
        hq = jnp.dot(fq.astype(bf), w2b, preferred_element_type=jnp.float32)

        for c in range(nck):
            off = c * ck
            kx = pk_ref[0, 0:1, pl.ds(off, ck)]
            ky = pk_ref[0, 1:2, pl.ds(off, ck)]
            kz = pk_ref[0, 2:3, pl.ds(off, ck)]
            pkc = pk_ref[0, :, pl.ds(off, ck)].astype(bf)
            cross = jnp.dot(pqtb, pkc, preferred_element_type=jnp.float32)
            kk = (kx * kx + ky * ky) + kz * kz
            dscr[:, pl.ds(off, ck)] = (kq - 2.0 * cross) + kk

        def step(_, sc):
            s1, s2, mx, mn = sc
            rmin = jnp.full((tq, 1), _INF, jnp.float32)
            rarg = jnp.zeros((tq, 1), jnp.int32)
            for c in range(nck):
                off = c * ck
                dch = dscr[:, pl.ds(off, ck)]
                cmin = jnp.min(dch, axis=1, keepdims=True)
                iota = jax.lax.broadcasted_iota(jnp.int32, (tq, ck), 1) + off
                carg = jnp.min(jnp.where(dch == cmin, iota, nk),
                               axis=1, keepdims=True)
                take = cmin < rmin
                rarg = jnp.where(take, carg, rarg)
                rmin = jnp.where(take, cmin, rmin)
            grow = jnp.zeros((tq, cin), jnp.float32)
            for c in range(nck):
                off = c * ck
                dch = dscr[:, pl.ds(off, ck)]
                iota = jax.lax.broadcasted_iota(jnp.int32, (tq, ck), 1) + off
                eq = iota == rarg
                grow = grow + jnp.dot(eq.astype(jnp.float32), _fk_chunk(off),
                                      preferred_element_type=jnp.float32)
                dscr[:, pl.ds(off, ck)] = jnp.where(eq, _INF, dch)
            diff = (grow - fq).astype(bf)
            out_t = jnp.dot(diff, w1b,
                            preferred_element_type=jnp.float32) + hq
            return (s1 + out_t, s2 + out_t * out_t,
                    jnp.maximum(mx, out_t), jnp.minimum(mn, out_t))

        init = (jnp.zeros((tq, cout), jnp.float32),
                jnp.zeros((tq, cout), jnp.float32),
                jnp.full((tq, cout), -_INF, jnp.float32),
                jnp.full((tq, cout), _INF, jnp.float32))
        s1, s2, mx, mn = jax.lax.fori_loop(0, _K, step, init)
        amax_scr[pl.ds(qoff, tq), :] = mx
        amin_scr[pl.ds(qoff, tq), :] = mn
        ts = jnp.sum(s1, axis=0, keepdims=True)
        tq_sq = jnp.sum(s2, axis=0, keepdims=True)
        return (s_sum + ts, s_sq + tq_sq)

    zc = jnp.zeros((1, cout), jnp.float32)
    s_sum, s_sq = jax.lax.fori_loop(0, ntiles, tile_body, (zc, zc))

    gmask = (jax.lax.broadcasted_iota(jnp.int32, (cout, _GROUPS), 0) // cg
             == jax.lax.broadcasted_iota(jnp.int32, (cout, _GROUPS), 1)
             ).astype(jnp.float32)
    gmask2 = (jax.lax.broadcasted_iota(jnp.int32, (_GROUPS, cout), 0)
              == jax.lax.broadcasted_iota(jnp.int32, (_GROUPS, cout), 1) // cg
              ).astype(jnp.float32)
    cnt = float(nq * _K * cg)
    sg = jnp.dot(s_sum, gmask, preferred_element_type=jnp.float32) / cnt
    qg = jnp.dot(s_sq, gmask, preferred_element_type=jnp.float32) / cnt
    vg = qg - sg * sg
    mean_c = jnp.dot(sg, gmask2, preferred_element_type=jnp.float32)
    var_c = jnp.dot(vg, gmask2, preferred_element_type=jnp.float32)
    denom = jnp.sqrt(var_c + _EPS)
    gam = g_ref[0:1, :]
    bet = b_ref[0:1, :]

    for t in range(ntiles):
        qoff = t * tq
        a_hi = amax_scr[pl.ds(qoff, tq), :]
        a_lo = amin_scr[pl.ds(qoff, tq), :]
        sel = jnp.where(gam >= 0.0, a_hi, a_lo)
        yv = (sel - mean_c) / denom
        yv = yv * gam + bet
        out_ref[0, pl.ds(qoff, tq), :] = jnp.maximum(yv, 0.0)


def _edge_layer(pqt, pk, wt, g, b, *, mode, fk=None, qidx=None, fq=None,
                stemw=None, stemb=None, tq=256, ck=1024):
    bsz, nq, _ = pqt.shape
    nk = pk.shape[2]
    cin = wt.shape[0] // 2
    cout = wt.shape[1]
    tq = min(tq, nq)
    ck = min(ck, nk)
    kern = functools.partial(_edge_body, mode=mode, nq=nq, nk=nk,
                             cin=cin, cout=cout, tq=tq, ck=ck)
    full2 = lambda b_: (0, 0)
    operands = [pqt, pk]
    in_specs = [pl.BlockSpec((1, nq, 3), lambda b_: (b_, 0, 0)),
                pl.BlockSpec((1, 3, nk), lambda b_: (b_, 0, 0))]
    if mode == "l1":
        operands += [stemw, stemb]
        in_specs += [pl.BlockSpec((3, stemw.shape[1]), full2),
                     pl.BlockSpec((1, stemb.shape[1]), full2)]
    elif mode == "gather":
        operands += [fk, qidx]
        in_specs += [pl.BlockSpec((1, nk, cin), lambda b_: (b_, 0, 0)),
                     pl.BlockSpec((1, nq, 1), lambda b_: (b_, 0, 0))]
    else:
        operands += [fk, fq]
        in_specs += [pl.BlockSpec((1, nk, cin), lambda b_: (b_, 0, 0)),
                     pl.BlockSpec((1, nq, cin), lambda b_: (b_, 0, 0))]
    operands += [wt, g, b]
    in_specs += [pl.BlockSpec((2 * cin, cout), full2),
                 pl.BlockSpec((1, cout), full2),
                 pl.BlockSpec((1, cout), full2)]
    return pl.pallas_call(
        kern,
        grid=(bsz,),
        in_specs=in_specs,
        out_specs=pl.BlockSpec((1, nq, cout), lambda b_: (b_, 0, 0)),
        out_shape=jax.ShapeDtypeStruct((bsz, nq, cout), jnp.float32),
        scratch_shapes=[
            pltpu.VMEM((tq, nk), jnp.float32),
            pltpu.VMEM((nk, cin), jnp.float32),
            pltpu.VMEM((nq, cout), jnp.float32),
            pltpu.VMEM((nq, cout), jnp.float32),
        ],
    )(*operands)


def kernel(xyz, stem_W, stem_b, l1_W, l1_g, l1_b, l2_W, l2_g, l2_b,
           l3_W, l3_g, l3_b, l4_W, l4_g, l4_b):
    bsz, _, n0 = xyz.shape
    n1 = max(1, int(n0 * 0.25))
    n2 = max(1, int(n1 * 0.25))

    p0 = xyz
    p0t = jnp.transpose(xyz, (0, 2, 1))
    stemw = jnp.transpose(stem_W)
    stemb = jnp.reshape(stem_b, (1, -1))
    row = lambda v: jnp.reshape(v, (1, -1))

    f0a = _edge_layer(p0t, p0, jnp.transpose(l1_W), row(l1_g), row(l1_b),
                      mode="l1", stemw=stemw, stemb=stemb)

    idx1, p1t = _fps(p0, n1)
    p1 = jnp.transpose(p1t, (0, 2, 1))

    f1a = _edge_layer(p1t, p0, jnp.transpose(l2_W), row(l2_g), row(l2_b),
                      mode="gather", fk=f0a, qidx=idx1)

    idx2, p2t = _fps(p1, n2)

    f2_mid = _edge_layer(p2t, p1, jnp.transpose(l3_W), row(l3_g), row(l3_b),
                         mode="gather", fk=f1a, qidx=idx2)
    f2a = _edge_layer(p2t, p1, jnp.transpose(l4_W), row(l4_g), row(l4_b),
                      mode="direct", fk=f1a, fq=f2_mid)
    return jnp.transpose(f2a, (0, 2, 1))


# speedup vs baseline: 2.0917x; 1.0489x over previous
import functools
from typing import Optional

import jax
import jax.numpy as jnp
from jax.experimental import pallas as pl
from jax.experimental.pallas import tpu as pltpu

_K = 16
_GROUPS = 8
_EPS = 1e-5
_INF = float("inf")


def _fps_body(pts_ref, idx_ref, pt_ref, *, n, m, rows):
    x = pts_ref[0, 0]
    y = pts_ref[0, 1]
    z = pts_ref[0, 2]
    fi = (jax.lax.broadcasted_iota(jnp.int32, (rows, 128), 0) * 128
          + jax.lax.broadcasted_iota(jnp.int32, (rows, 128), 1))
    px0 = x[0, 0]
    py0 = y[0, 0]
    pz0 = z[0, 0]
    dx = x - px0
    dy = y - py0
    dz = z - pz0
    d0 = (dx * dx + dy * dy) + dz * dz
    idx_ref[0, 0:1, 0:1] = jnp.zeros((1, 1), jnp.int32)
    pt_ref[0, 0:1, 0:1] = jnp.reshape(px0, (1, 1))
    pt_ref[0, 0:1, 1:2] = jnp.reshape(py0, (1, 1))
    pt_ref[0, 0:1, 2:3] = jnp.reshape(pz0, (1, 1))

    def body(i, d):
        rmax = jnp.max(d)
        nxt = jnp.min(jnp.where(d == rmax, fi, n))
        sel = fi == nxt
        px = jnp.sum(jnp.where(sel, x, 0.0))
        py = jnp.sum(jnp.where(sel, y, 0.0))
        pz = jnp.sum(jnp.where(sel, z, 0.0))
        idx_ref[0, pl.ds(i, 1), 0:1] = jnp.reshape(nxt, (1, 1))
        pt_ref[0, pl.ds(i, 1), 0:1] = jnp.reshape(px, (1, 1))
        pt_ref[0, pl.ds(i, 1), 1:2] = jnp.reshape(py, (1, 1))
        pt_ref[0, pl.ds(i, 1), 2:3] = jnp.reshape(pz, (1, 1))
        ddx = x - px
        ddy = y - py
        ddz = z - pz
        nd = (ddx * ddx + ddy * ddy) + ddz * ddz
        return jnp.minimum(d, nd)

    jax.lax.fori_loop(1, m, body, d0)


def _fps(pts, m):
    bsz, _, n = pts.shape
    rows = n // 128
    pts4 = jnp.reshape(pts, (bsz, 3, rows, 128))
    kern = functools.partial(_fps_body, n=n, m=m, rows=rows)
    return pl.pallas_call(
        kern,
        grid=(bsz,),
        in_specs=[pl.BlockSpec((1, 3, rows, 128), lambda b: (b, 0, 0, 0))],
        out_specs=[
            pl.BlockSpec((1, m, 1), lambda b: (b, 0, 0)),
            pl.BlockSpec((1, m, 3), lambda b: (b, 0, 0)),
        ],
        out_shape=[
            jax.ShapeDtypeStruct((bsz, m, 1), jnp.int32),
            jax.ShapeDtypeStruct((bsz, m, 3), jnp.float32),
        ],
    )(pts4)


def _edge_body(*refs, mode, nq, nk, cin, cout, tq, ck):
    if mode == "l1":
        (pqt_ref, pk_ref, stemw_ref, stemb_ref, wt_ref, g_ref, b_ref,
         out_ref, dscr, fkscr, amax_scr, amin_scr) = refs
    elif mode == "gather":
        (pqt_ref, pk_ref, fk_ref, qidx_ref, wt_ref, g_ref, b_ref,
         out_ref, dscr, fkscr, amax_scr, amin_scr) = refs
    else:
        (pqt_ref, pk_ref, fk_ref, fq_ref, wt_ref, g_ref, b_ref,
         out_ref, dscr, fkscr, amax_scr, amin_scr) = refs

    nck = nk // ck
    ntiles = nq // tq
    cg = cout // _GROUPS
    bf = jnp.bfloat16
    w1b = wt_ref[0:cin, :].astype(bf)
    w2b = wt_ref[cin:2 * cin, :].astype(bf)
    if mode == "l1":
        swb = stemw_ref[...].astype(bf)
        sb = stemb_ref[0:1, :]

    def _stem(pts_t):
        return jnp.dot(pts_t.astype(bf), swb,
                       preferred_element_type=jnp.float32) + sb

    def _fk_chunk(off):
        if mode == "l1":
            return fkscr[pl.ds(off, ck), :]
        return fk_ref[0, pl.ds(off, ck), :]

    if mode == "l1":
        for c in range(nck):
            off = c * ck
            fkscr[pl.ds(off, ck), :] = _stem(pqt_ref[0, pl.ds(off, ck), :])

    def tile_body(t, carry):
        s_sum, s_sq = carry
        qoff = t * tq
        pqt = pqt_ref[0, pl.ds(qoff, tq), :]
        qx = pqt[:, 0:1]
        qy = pqt[:, 1:2]
        qz = pqt[:, 2:3]
        kq = (qx * qx + qy * qy) + qz * qz
        pqtb = pqt.astype(bf)

        if mode == "l1":
            fq = _stem(pqt)
        elif mode == "gather":
            qid = qidx_ref[0, pl.ds(qoff, tq), :]
            fq = jnp.zeros((tq, cin), jnp.float32)
            for c in range(nck):
                off = c * ck
                iota = jax.lax.broadcasted_iota(jnp.int32, (tq, ck), 1) + off
                oh = (iota == qid).astype(jnp.float32)
                fq = fq + jnp.dot(oh, fk_ref[0, pl.ds(off, ck), :],
                                  preferred_element_type=jnp.float32)
        else:
            fq = fq_ref[0, pl.ds(qoff, tq), :]
        hq = jnp.dot(fq.astype(bf), w2b, preferred_element_type=jnp.float32)

        for c in range(nck):
            off = c * ck
            kx = pk_ref[0, 0:1, pl.ds(off, ck)]
            ky = pk_ref[0, 1:2, pl.ds(off, ck)]
            kz = pk_ref[0, 2:3, pl.ds(off, ck)]
            pkc = pk_ref[0, :, pl.ds(off, ck)].astype(bf)
            cross = jnp.dot(pqtb, pkc, preferred_element_type=jnp.float32)
            kk = (kx * kx + ky * ky) + kz * kz
            dscr[:, pl.ds(off, ck)] = (kq - 2.0 * cross) + kk

        def step(_, sc):
            s1, s2, mx, mn = sc
            rmin = jnp.full((tq, 1), _INF, jnp.float32)
            rarg = jnp.zeros((tq, 1), jnp.int32)
            for c in range(nck):
                off = c * ck
                dch = dscr[:, pl.ds(off, ck)]
                cmin = jnp.min(dch, axis=1, keepdims=True)
                iota = jax.lax.broadcasted_iota(jnp.int32, (tq, ck), 1) + off
                carg = jnp.min(jnp.where(dch == cmin, iota, nk),
                               axis=1, keepdims=True)
                take = cmin < rmin
                rarg = jnp.where(take, carg, rarg)
                rmin = jnp.where(take, cmin, rmin)
            grow = jnp.zeros((tq, cin), jnp.float32)
            for c in range(nck):
                off = c * ck
                dch = dscr[:, pl.ds(off, ck)]
                iota = jax.lax.broadcasted_iota(jnp.int32, (tq, ck), 1) + off
                eq = iota == rarg
                grow = grow + jnp.dot(eq.astype(jnp.float32), _fk_chunk(off),
                                      preferred_element_type=jnp.float32)
                dscr[:, pl.ds(off, ck)] = jnp.where(eq, _INF, dch)
            diff = (grow - fq).astype(bf)
            out_t = jnp.dot(diff, w1b,
                            preferred_element_type=jnp.float32) + hq
            return (s1 + out_t, s2 + out_t * out_t,
                    jnp.maximum(mx, out_t), jnp.minimum(mn, out_t))

        init = (jnp.zeros((tq, cout), jnp.float32),
                jnp.zeros((tq, cout), jnp.float32),
                jnp.full((tq, cout), -_INF, jnp.float32),
                jnp.full((tq, cout), _INF, jnp.float32))
        s1, s2, mx, mn = jax.lax.fori_loop(0, _K, step, init)
        amax_scr[pl.ds(qoff, tq), :] = mx
        amin_scr[pl.ds(qoff, tq), :] = mn
        ts = jnp.sum(s1, axis=0, keepdims=True)
        tq_sq = jnp.sum(s2, axis=0, keepdims=True)
        return (s_sum + ts, s_sq + tq_sq)

    zc = jnp.zeros((1, cout), jnp.float32)
    s_sum, s_sq = jax.lax.fori_loop(0, ntiles, tile_body, (zc, zc))

    gmask = (jax.lax.broadcasted_iota(jnp.int32, (cout, _GROUPS), 0) // cg
             == jax.lax.broadcasted_iota(jnp.int32, (cout, _GROUPS), 1)
             ).astype(jnp.float32)
    gmask2 = (jax.lax.broadcasted_iota(jnp.int32, (_GROUPS, cout), 0)
              == jax.lax.broadcasted_iota(jnp.int32, (_GROUPS, cout), 1) // cg
              ).astype(jnp.float32)
    cnt = float(nq * _K * cg)
    sg = jnp.dot(s_sum, gmask, preferred_element_type=jnp.float32) / cnt
    qg = jnp.dot(s_sq, gmask, preferred_element_type=jnp.float32) / cnt
    vg = qg - sg * sg
    mean_c = jnp.dot(sg, gmask2, preferred_element_type=jnp.float32)
    var_c = jnp.dot(vg, gmask2, preferred_element_type=jnp.float32)
    denom = jnp.sqrt(var_c + _EPS)
    gam = g_ref[0:1, :]
    bet = b_ref[0:1, :]

    for t in range(ntiles):
        qoff = t * tq
        a_hi = amax_scr[pl.ds(qoff, tq), :]
        a_lo = amin_scr[pl.ds(qoff, tq), :]
        sel = jnp.where(gam >= 0.0, a_hi, a_lo)
        yv = (sel - mean_c) / denom
        yv = yv * gam + bet
        out_ref[0, pl.ds(qoff, tq), :] = jnp.maximum(yv, 0.0)


def _edge_layer(pqt, pk, wt, g, b, *, mode, fk=None, qidx=None, fq=None,
                stemw=None, stemb=None, tq=512, ck=1024):
    bsz, nq, _ = pqt.shape
    nk = pk.shape[2]
    cin = wt.shape[0] // 2
    cout = wt.shape[1]
    tq = min(tq, nq)
    ck = min(ck, nk)
    kern = functools.partial(_edge_body, mode=mode, nq=nq, nk=nk,
                             cin=cin, cout=cout, tq=tq, ck=ck)
    full2 = lambda b_: (0, 0)
    operands = [pqt, pk]
    in_specs = [pl.BlockSpec((1, nq, 3), lambda b_: (b_, 0, 0)),
                pl.BlockSpec((1, 3, nk), lambda b_: (b_, 0, 0))]
    if mode == "l1":
        operands += [stemw, stemb]
        in_specs += [pl.BlockSpec((3, stemw.shape[1]), full2),
                     pl.BlockSpec((1, stemb.shape[1]), full2)]
    elif mode == "gather":
        operands += [fk, qidx]
        in_specs += [pl.BlockSpec((1, nk, cin), lambda b_: (b_, 0, 0)),
                     pl.BlockSpec((1, nq, 1), lambda b_: (b_, 0, 0))]
    else:
        operands += [fk, fq]
        in_specs += [pl.BlockSpec((1, nk, cin), lambda b_: (b_, 0, 0)),
                     pl.BlockSpec((1, nq, cin), lambda b_: (b_, 0, 0))]
    operands += [wt, g, b]
    in_specs += [pl.BlockSpec((2 * cin, cout), full2),
                 pl.BlockSpec((1, cout), full2),
                 pl.BlockSpec((1, cout), full2)]
    return pl.pallas_call(
        kern,
        grid=(bsz,),
        in_specs=in_specs,
        out_specs=pl.BlockSpec((1, nq, cout), lambda b_: (b_, 0, 0)),
        out_shape=jax.ShapeDtypeStruct((bsz, nq, cout), jnp.float32),
        scratch_shapes=[
            pltpu.VMEM((tq, nk), jnp.float32),
            pltpu.VMEM((nk, cin), jnp.float32),
            pltpu.VMEM((nq, cout), jnp.float32),
            pltpu.VMEM((nq, cout), jnp.float32),
        ],
    )(*operands)


def kernel(xyz, stem_W, stem_b, l1_W, l1_g, l1_b, l2_W, l2_g, l2_b,
           l3_W, l3_g, l3_b, l4_W, l4_g, l4_b):
    bsz, _, n0 = xyz.shape
    n1 = max(1, int(n0 * 0.25))
    n2 = max(1, int(n1 * 0.25))

    p0 = xyz
    p0t = jnp.transpose(xyz, (0, 2, 1))
    stemw = jnp.transpose(stem_W)
    stemb = jnp.reshape(stem_b, (1, -1))
    row = lambda v: jnp.reshape(v, (1, -1))

    f0a = _edge_layer(p0t, p0, jnp.transpose(l1_W), row(l1_g), row(l1_b),
                      mode="l1", stemw=stemw, stemb=stemb)

    idx1, p1t = _fps(p0, n1)
    p1 = jnp.transpose(p1t, (0, 2, 1))

    f1a = _edge_layer(p1t, p0, jnp.transpose(l2_W), row(l2_g), row(l2_b),
                      mode="gather", fk=f0a, qidx=idx1)

    idx2, p2t = _fps(p1, n2)

    f2_mid = _edge_layer(p2t, p1, jnp.transpose(l3_W), row(l3_g), row(l3_b),
                         mode="gather", fk=f1a, qidx=idx2)
    f2a = _edge_layer(p2t, p1, jnp.transpose(l4_W), row(l4_g), row(l4_b),
                      mode="direct", fk=f1a, fq=f2_mid)
    return jnp.transpose(f2a, (0, 2, 1))


# speedup vs baseline: 2.1311x; 1.0188x over previous
import functools
from typing import Optional

import jax
import jax.numpy as jnp
from jax.experimental import pallas as pl
from jax.experimental.pallas import tpu as pltpu

_K = 16
_GROUPS = 8
_EPS = 1e-5
_INF = float("inf")


def _fps_body(pts_ref, idx_ref, pt_ref, *, n, m, rows):
    x = pts_ref[0, 0]
    y = pts_ref[0, 1]
    z = pts_ref[0, 2]
    fi = (jax.lax.broadcasted_iota(jnp.int32, (rows, 128), 0) * 128
          + jax.lax.broadcasted_iota(jnp.int32, (rows, 128), 1))
    px0 = x[0, 0]
    py0 = y[0, 0]
    pz0 = z[0, 0]
    dx = x - px0
    dy = y - py0
    dz = z - pz0
    d0 = (dx * dx + dy * dy) + dz * dz
    idx_ref[0, 0:1, 0:1] = jnp.zeros((1, 1), jnp.int32)
    pt_ref[0, 0:1, 0:1] = jnp.reshape(px0, (1, 1))
    pt_ref[0, 0:1, 1:2] = jnp.reshape(py0, (1, 1))
    pt_ref[0, 0:1, 2:3] = jnp.reshape(pz0, (1, 1))

    def body(i, d):
        rmax = jnp.max(d)
        nxt = jnp.min(jnp.where(d == rmax, fi, n))
        sel = fi == nxt
        px = jnp.sum(jnp.where(sel, x, 0.0))
        py = jnp.sum(jnp.where(sel, y, 0.0))
        pz = jnp.sum(jnp.where(sel, z, 0.0))
        idx_ref[0, pl.ds(i, 1), 0:1] = jnp.reshape(nxt, (1, 1))
        pt_ref[0, pl.ds(i, 1), 0:1] = jnp.reshape(px, (1, 1))
        pt_ref[0, pl.ds(i, 1), 1:2] = jnp.reshape(py, (1, 1))
        pt_ref[0, pl.ds(i, 1), 2:3] = jnp.reshape(pz, (1, 1))
        ddx = x - px
        ddy = y - py
        ddz = z - pz
        nd = (ddx * ddx + ddy * ddy) + ddz * ddz
        return jnp.minimum(d, nd)

    jax.lax.fori_loop(1, m, body, d0)


def _fps(pts, m):
    bsz, _, n = pts.shape
    rows = n // 128
    pts4 = jnp.reshape(pts, (bsz, 3, rows, 128))
    kern = functools.partial(_fps_body, n=n, m=m, rows=rows)
    return pl.pallas_call(
        kern,
        grid=(bsz,),
        in_specs=[pl.BlockSpec((1, 3, rows, 128), lambda b: (b, 0, 0, 0))],
        out_specs=[
            pl.BlockSpec((1, m, 1), lambda b: (b, 0, 0)),
            pl.BlockSpec((1, m, 3), lambda b: (b, 0, 0)),
        ],
        out_shape=[
            jax.ShapeDtypeStruct((bsz, m, 1), jnp.int32),
            jax.ShapeDtypeStruct((bsz, m, 3), jnp.float32),
        ],
    )(pts4)


def _edge_body(*refs, mode, nq, nk, cin, cout, tq, ck):
    if mode == "l1":
        (pqt_ref, pk_ref, stemw_ref, stemb_ref, wt_ref, g_ref, b_ref,
         out_ref, dscr, fkscr, amax_scr, amin_scr) = refs
    elif mode == "gather":
        (pqt_ref, pk_ref, fk_ref, qidx_ref, wt_ref, g_ref, b_ref,
         out_ref, dscr, fkscr, amax_scr, amin_scr) = refs
    else:
        (pqt_ref, pk_ref, fk_ref, fq_ref, wt_ref, g_ref, b_ref,
         out_ref, dscr, fkscr, amax_scr, amin_scr) = refs

    nck = nk // ck
    ntiles = nq // tq
    cg = cout // _GROUPS
    bf = jnp.bfloat16
    w1b = wt_ref[0:cin, :].astype(bf)
    w2b = wt_ref[cin:2 * cin, :].astype(bf)
    if mode == "l1":
        swb = stemw_ref[...].astype(bf)
        sb = stemb_ref[0:1, :]

    def _stem(pts_t):
        return jnp.dot(pts_t.astype(bf), swb,
                       preferred_element_type=jnp.float32) + sb

    def _fk_chunk(off):
        if mode == "l1":
            return fkscr[pl.ds(off, ck), :]
        return fk_ref[0, pl.ds(off, ck), :]

    if mode == "l1":
        for c in range(nck):
            off = c * ck
            fkscr[pl.ds(off, ck), :] = _stem(pqt_ref[0, pl.ds(off, ck), :])

    def tile_body(t, carry):
        s_sum, s_sq = carry
        qoff = t * tq
        pqt = pqt_ref[0, pl.ds(qoff, tq), :]
        qx = pqt[:, 0:1]
        qy = pqt[:, 1:2]
        qz = pqt[:, 2:3]
        kq = (qx * qx + qy * qy) + qz * qz
        pqtb = pqt.astype(bf)

        if mode == "l1":
            fq = _stem(pqt)
        elif mode == "gather":
            qid = qidx_ref[0, pl.ds(qoff, tq), :]
            fq = jnp.zeros((tq, cin), jnp.float32)
            for c in range(nck):
                off = c * ck
                iota = jax.lax.broadcasted_iota(jnp.int32, (tq, ck), 1) + off
                oh = (iota == qid).astype(jnp.float32)
                fq = fq + jnp.dot(oh, fk_ref[0, pl.ds(off, ck), :],
                                  preferred_element_type=jnp.float32)
        else:
            fq = fq_ref[0, pl.ds(qoff, tq), :]
        hq = jnp.dot(fq.astype(bf), w2b, preferred_element_type=jnp.float32)

        for c in range(nck):
            off = c * ck
            kx = pk_ref[0, 0:1, pl.ds(off, ck)]
            ky = pk_ref[0, 1:2, pl.ds(off, ck)]
            kz = pk_ref[0, 2:3, pl.ds(off, ck)]
            pkc = pk_ref[0, :, pl.ds(off, ck)].astype(bf)
            cross = jnp.dot(pqtb, pkc, preferred_element_type=jnp.float32)
            kk = (kx * kx + ky * ky) + kz * kz
            dscr[:, pl.ds(off, ck)] = (kq - 2.0 * cross) + kk

        def step(_, sc):
            s1, s2, mx, mn = sc
            rmin = jnp.full((tq, 1), _INF, jnp.float32)
            rarg = jnp.zeros((tq, 1), jnp.int32)
            for c in range(nck):
                off = c * ck
                dch = dscr[:, pl.ds(off, ck)]
                cmin = jnp.min(dch, axis=1, keepdims=True)
                iota = jax.lax.broadcasted_iota(jnp.int32, (tq, ck), 1) + off
                carg = jnp.min(jnp.where(dch == cmin, iota, nk),
                               axis=1, keepdims=True)
                take = cmin < rmin
                rarg = jnp.where(take, carg, rarg)
                rmin = jnp.where(take, cmin, rmin)
            grow = jnp.zeros((tq, cin), jnp.float32)
            for c in range(nck):
                off = c * ck
                dch = dscr[:, pl.ds(off, ck)]
                iota = jax.lax.broadcasted_iota(jnp.int32, (tq, ck), 1) + off
                eq = iota == rarg
                grow = grow + jnp.dot(eq.astype(jnp.float32), _fk_chunk(off),
                                      preferred_element_type=jnp.float32)
                dscr[:, pl.ds(off, ck)] = jnp.where(eq, _INF, dch)
            diff = (grow - fq).astype(bf)
            out_t = jnp.dot(diff, w1b,
                            preferred_element_type=jnp.float32) + hq
            return (s1 + out_t, s2 + out_t * out_t,
                    jnp.maximum(mx, out_t), jnp.minimum(mn, out_t))

        init = (jnp.zeros((tq, cout), jnp.float32),
                jnp.zeros((tq, cout), jnp.float32),
                jnp.full((tq, cout), -_INF, jnp.float32),
                jnp.full((tq, cout), _INF, jnp.float32))
        s1, s2, mx, mn = jax.lax.fori_loop(0, _K, step, init)
        amax_scr[pl.ds(qoff, tq), :] = mx
        amin_scr[pl.ds(qoff, tq), :] = mn
        ts = jnp.sum(s1, axis=0, keepdims=True)
        tq_sq = jnp.sum(s2, axis=0, keepdims=True)
        return (s_sum + ts, s_sq + tq_sq)

    zc = jnp.zeros((1, cout), jnp.float32)
    s_sum, s_sq = jax.lax.fori_loop(0, ntiles, tile_body, (zc, zc))

    gmask = (jax.lax.broadcasted_iota(jnp.int32, (cout, _GROUPS), 0) // cg
             == jax.lax.broadcasted_iota(jnp.int32, (cout, _GROUPS), 1)
             ).astype(jnp.float32)
    gmask2 = (jax.lax.broadcasted_iota(jnp.int32, (_GROUPS, cout), 0)
              == jax.lax.broadcasted_iota(jnp.int32, (_GROUPS, cout), 1) // cg
              ).astype(jnp.float32)
    cnt = float(nq * _K * cg)
    sg = jnp.dot(s_sum, gmask, preferred_element_type=jnp.float32) / cnt
    qg = jnp.dot(s_sq, gmask, preferred_element_type=jnp.float32) / cnt
    vg = qg - sg * sg
    mean_c = jnp.dot(sg, gmask2, preferred_element_type=jnp.float32)
    var_c = jnp.dot(vg, gmask2, preferred_element_type=jnp.float32)
    denom = jnp.sqrt(var_c + _EPS)
    gam = g_ref[0:1, :]
    bet = b_ref[0:1, :]

    for t in range(ntiles):
        qoff = t * tq
        a_hi = amax_scr[pl.ds(qoff, tq), :]
        a_lo = amin_scr[pl.ds(qoff, tq), :]
        sel = jnp.where(gam >= 0.0, a_hi, a_lo)
        yv = (sel - mean_c) / denom
        yv = yv * gam + bet
        out_ref[0, pl.ds(qoff, tq), :] = jnp.maximum(yv, 0.0)


def _edge_layer(pqt, pk, wt, g, b, *, mode, fk=None, qidx=None, fq=None,
                stemw=None, stemb=None, tq=512, ck=2048):
    bsz, nq, _ = pqt.shape
    nk = pk.shape[2]
    cin = wt.shape[0] // 2
    cout = wt.shape[1]
    tq = min(tq, nq)
    ck = min(ck, nk)
    kern = functools.partial(_edge_body, mode=mode, nq=nq, nk=nk,
                             cin=cin, cout=cout, tq=tq, ck=ck)
    full2 = lambda b_: (0, 0)
    operands = [pqt, pk]
    in_specs = [pl.BlockSpec((1, nq, 3), lambda b_: (b_, 0, 0)),
                pl.BlockSpec((1, 3, nk), lambda b_: (b_, 0, 0))]
    if mode == "l1":
        operands += [stemw, stemb]
        in_specs += [pl.BlockSpec((3, stemw.shape[1]), full2),
                     pl.BlockSpec((1, stemb.shape[1]), full2)]
    elif mode == "gather":
        operands += [fk, qidx]
        in_specs += [pl.BlockSpec((1, nk, cin), lambda b_: (b_, 0, 0)),
                     pl.BlockSpec((1, nq, 1), lambda b_: (b_, 0, 0))]
    else:
        operands += [fk, fq]
        in_specs += [pl.BlockSpec((1, nk, cin), lambda b_: (b_, 0, 0)),
                     pl.BlockSpec((1, nq, cin), lambda b_: (b_, 0, 0))]
    operands += [wt, g, b]
    in_specs += [pl.BlockSpec((2 * cin, cout), full2),
                 pl.BlockSpec((1, cout), full2),
                 pl.BlockSpec((1, cout), full2)]
    return pl.pallas_call(
        kern,
        grid=(bsz,),
        in_specs=in_specs,
        out_specs=pl.BlockSpec((1, nq, cout), lambda b_: (b_, 0, 0)),
        out_shape=jax.ShapeDtypeStruct((bsz, nq, cout), jnp.float32),
        scratch_shapes=[
            pltpu.VMEM((tq, nk), jnp.float32),
            pltpu.VMEM((nk, cin), jnp.float32),
            pltpu.VMEM((nq, cout), jnp.float32),
            pltpu.VMEM((nq, cout), jnp.float32),
        ],
    )(*operands)


def kernel(xyz, stem_W, stem_b, l1_W, l1_g, l1_b, l2_W, l2_g, l2_b,
           l3_W, l3_g, l3_b, l4_W, l4_g, l4_b):
    bsz, _, n0 = xyz.shape
    n1 = max(1, int(n0 * 0.25))
    n2 = max(1, int(n1 * 0.25))

    p0 = xyz
    p0t = jnp.transpose(xyz, (0, 2, 1))
    stemw = jnp.transpose(stem_W)
    stemb = jnp.reshape(stem_b, (1, -1))
    row = lambda v: jnp.reshape(v, (1, -1))

    f0a = _edge_layer(p0t, p0, jnp.transpose(l1_W), row(l1_g), row(l1_b),
                      mode="l1", stemw=stemw, stemb=stemb)

    idx1, p1t = _fps(p0, n1)
    p1 = jnp.transpose(p1t, (0, 2, 1))

    f1a = _edge_layer(p1t, p0, jnp.transpose(l2_W), row(l2_g), row(l2_b),
                      mode="gather", fk=f0a, qidx=idx1)

    idx2, p2t = _fps(p1, n2)

    f2_mid = _edge_layer(p2t, p1, jnp.transpose(l3_W), row(l3_g), row(l3_b),
                         mode="gather", fk=f1a, qidx=idx2)
    f2a = _edge_layer(p2t, p1, jnp.transpose(l4_W), row(l4_g), row(l4_b),
                      mode="direct", fk=f1a, fq=f2_mid)
    return jnp.transpose(f2a, (0, 2, 1))
